# baseline (speedup 1.0000x reference)
import functools

import jax
import jax.numpy as jnp
import numpy as np
from jax import lax
from jax.experimental import pallas as pl
from jax.experimental.pallas import tpu as pltpu
from jax.experimental.pallas import tpu_sc as plsc

_VOCAB = 1000000
_D = 64
_S = 200
_NC = 2
_NS = 16
_NW = _NC * _NS
_CH = 40
_NCH = _S // _CH


def _positional_encoding(n=10000.0):
    position = np.arange(_S, dtype=np.float32)[:, None]
    division_term = np.exp(
        np.arange(0, _D, 2, dtype=np.float32) * (-np.log(n) / _D)
    )
    pos_enc = np.zeros((_S, _D), dtype=np.float32)
    pos_enc[:, 0::2] = np.sin(position * division_term)
    pos_enc[:, 1::2] = np.cos(position * division_term)
    return jnp.asarray(pos_enc)


def _sc_body(x_hbm, tab_hbm, pos_hbm, out_hbm, idx_v, rows_v, pos_v, sem):
    wid = lax.axis_index("s") * _NC + lax.axis_index("c")
    seq_per_w = x_hbm.shape[0] // (_S * _NW)
    pltpu.sync_copy(pos_hbm, pos_v)

    def per_seq(s, carry):
        base = (wid * seq_per_w + s) * _S
        pltpu.sync_copy(x_hbm.at[pl.ds(base, _S)], idx_v)
        copies = []
        for c in range(_NCH):
            copies.append(
                pltpu.async_copy(
                    tab_hbm.at[idx_v.at[pl.ds(c * _CH, _CH)]],
                    rows_v.at[pl.ds(c * _CH, _CH)],
                    sem,
                )
            )
        for cp in copies:
            cp.wait()

        def add_row(r, carry2):
            for g in range(_D // 16):
                sl = pl.ds(g * 16, 16)
                rows_v[r, sl] = rows_v[r, sl] + pos_v[r, sl]
            return carry2

        lax.fori_loop(0, _S, add_row, 0, unroll=2)
        pltpu.sync_copy(rows_v, out_hbm.at[pl.ds(base, _S)])
        return carry

    lax.fori_loop(0, seq_per_w, per_seq, 0)


def kernel(x, embedding_table):
    B, S = x.shape
    xf = x.reshape(B * S).astype(jnp.int32)
    pos = _positional_encoding()

    mesh = plsc.VectorSubcoreMesh(core_axis_name="c", subcore_axis_name="s")
    out = pl.kernel(
        _sc_body,
        out_type=jax.ShapeDtypeStruct((B * S, _D), jnp.float32),
        mesh=mesh,
        scratch_types=[
            pltpu.VMEM((_S,), jnp.int32),
            pltpu.VMEM((_S, _D), jnp.float32),
            pltpu.VMEM((_S, _D), jnp.float32),
            pltpu.SemaphoreType.DMA,
        ],
        compiler_params=pltpu.CompilerParams(use_tc_tiling_on_sc=False),
    )(xf, embedding_table, pos)
    return out.reshape(B, S, _D)


# speedup vs baseline: 1.2620x; 1.2620x over previous
import jax
import jax.numpy as jnp
import numpy as np
from jax import lax
from jax.experimental import pallas as pl
from jax.experimental.pallas import tpu as pltpu
from jax.experimental.pallas import tpu_sc as plsc

_D = 64
_S = 200
_NC = 2
_NS = 16
_NW = _NC * _NS
_CH = 40
_NCH = _S // _CH
_NBUF = 4
_LEAD = 2


def _positional_encoding(n=10000.0):
    position = np.arange(_S, dtype=np.float32)[:, None]
    division_term = np.exp(
        np.arange(0, _D, 2, dtype=np.float32) * (-np.log(n) / _D)
    )
    pos_enc = np.zeros((_S, _D), dtype=np.float32)
    pos_enc[:, 0::2] = np.sin(position * division_term)
    pos_enc[:, 1::2] = np.cos(position * division_term)
    return jnp.asarray(pos_enc)


def _sc_body(x_hbm, tab_hbm, pos_hbm, out_hbm, idx_v, rows_v, pos_v,
             gsem, ssem):
    wid = lax.axis_index("s") * _NC + lax.axis_index("c")
    spw = x_hbm.shape[0] // (_S * _NW)
    base0 = wid * spw * _S
    pltpu.sync_copy(pos_hbm, pos_v)

    def issue_gather(b, s):
        base = base0 + s * _S
        pltpu.sync_copy(x_hbm.at[pl.ds(base, _S)], idx_v.at[b])
        for c in range(_NCH):
            pltpu.async_copy(
                tab_hbm.at[idx_v.at[b, pl.ds(c * _CH, _CH)]],
                rows_v.at[b, pl.ds(c * _CH, _CH)],
                gsem.at[b],
            )

    def wait_gather(b):
        for c in range(_NCH):
            pltpu.make_async_copy(
                tab_hbm.at[idx_v.at[b, pl.ds(c * _CH, _CH)]],
                rows_v.at[b, pl.ds(c * _CH, _CH)],
                gsem.at[b],
            ).wait()

    def start_store(b, s):
        base = base0 + s * _S
        pltpu.async_copy(rows_v.at[b], out_hbm.at[pl.ds(base, _S)],
                         ssem.at[b])

    def wait_store(b, s):
        base = base0 + s * _S
        pltpu.make_async_copy(rows_v.at[b], out_hbm.at[pl.ds(base, _S)],
                              ssem.at[b]).wait()

    for b in range(_LEAD):
        issue_gather(b, b)

    def outer(io, carry):
        for b in range(_NBUF):
            s = io * _NBUF + b
            wait_gather(b)

            def add_row(r, c2):
                for g in range(_D // 16):
                    sl = pl.ds(g * 16, 16)
                    rows_v[b, r, sl] = rows_v[b, r, sl] + pos_v[r, sl]
                return c2

            lax.fori_loop(0, _S, add_row, 0, unroll=4)
            start_store(b, s)

            nxt = s + _LEAD
            bn = (b + _LEAD) % _NBUF

            @pl.when(nxt < spw)
            def _():
                @pl.when(nxt >= _NBUF)
                def _():
                    wait_store(bn, nxt - _NBUF)

                issue_gather(bn, nxt)

        return carry

    lax.fori_loop(0, spw // _NBUF, outer, 0)
    for b in range(_NBUF):
        wait_store(b, spw - _NBUF + b)


def kernel(x, embedding_table):
    B, S = x.shape
    xf = x.reshape(B * S).astype(jnp.int32)
    pos = _positional_encoding()

    mesh = plsc.VectorSubcoreMesh(core_axis_name="c", subcore_axis_name="s")
    out = pl.kernel(
        _sc_body,
        out_type=jax.ShapeDtypeStruct((B * S, _D), jnp.float32),
        mesh=mesh,
        scratch_types=[
            pltpu.VMEM((_NBUF, _S), jnp.int32),
            pltpu.VMEM((_NBUF, _S, _D), jnp.float32),
            pltpu.VMEM((_S, _D), jnp.float32),
            pltpu.SemaphoreType.DMA((_NBUF,)),
            pltpu.SemaphoreType.DMA((_NBUF,)),
        ],
        compiler_params=pltpu.CompilerParams(use_tc_tiling_on_sc=False),
    )(xf, embedding_table, pos)
    return out.reshape(B, S, _D)


# speedup vs baseline: 1.2626x; 1.0005x over previous
import jax
import jax.numpy as jnp
import numpy as np
from jax import lax
from jax.experimental import pallas as pl
from jax.experimental.pallas import tpu as pltpu
from jax.experimental.pallas import tpu_sc as plsc

_D = 64
_S = 200
_NC = 2
_NS = 16
_NW = _NC * _NS
_CH = 40
_NCH = _S // _CH
_NBUF = 4
_LEAD = 2


def _positional_encoding(n=10000.0):
    position = np.arange(_S, dtype=np.float32)[:, None]
    division_term = np.exp(
        np.arange(0, _D, 2, dtype=np.float32) * (-np.log(n) / _D)
    )
    pos_enc = np.zeros((_S, _D), dtype=np.float32)
    pos_enc[:, 0::2] = np.sin(position * division_term)
    pos_enc[:, 1::2] = np.cos(position * division_term)
    return jnp.asarray(pos_enc)


def _sc_body(x_hbm, tab_hbm, pos_hbm, out_hbm, idx_v, rows_v, pos_v,
             gsem, ssem):
    wid = lax.axis_index("s") * _NC + lax.axis_index("c")
    spw = x_hbm.shape[0] // (_S * _NW)
    base0 = wid * spw * _S
    tab2 = tab_hbm
    pltpu.sync_copy(pos_hbm, pos_v)

    def issue_gather(b, s):
        base = base0 + s * _S
        pltpu.sync_copy(x_hbm.at[pl.ds(base, _S)], idx_v.at[b])
        for c in range(_NCH):
            pltpu.async_copy(
                tab2.at[idx_v.at[b, pl.ds(c * _CH, _CH)]],
                rows_v.at[b, pl.ds(c * _CH, _CH)],
                gsem.at[b],
            )

    def wait_gather(b):
        for c in range(_NCH):
            pltpu.make_async_copy(
                tab2.at[idx_v.at[b, pl.ds(c * _CH, _CH)]],
                rows_v.at[b, pl.ds(c * _CH, _CH)],
                gsem.at[b],
            ).wait()

    def start_store(b, s):
        bi = wid * spw + s
        pltpu.async_copy(rows_v.at[b], out_hbm.at[bi], ssem.at[b])

    def wait_store(b, s):
        bi = wid * spw + s
        pltpu.make_async_copy(rows_v.at[b], out_hbm.at[bi],
                              ssem.at[b]).wait()

    for b in range(_LEAD):
        issue_gather(b, b)

    def outer(io, carry):
        for b in range(_NBUF):
            s = io * _NBUF + b
            wait_gather(b)

            def add_row(r, c2):
                for g in range(_D // 16):
                    sl = pl.ds(g * 16, 16)
                    rows_v[b, r, sl] = rows_v[b, r, sl] + pos_v[r, sl]
                return c2

            lax.fori_loop(0, _S, add_row, 0, unroll=4)
            start_store(b, s)

            nxt = s + _LEAD
            bn = (b + _LEAD) % _NBUF

            @pl.when(nxt < spw)
            def _():
                @pl.when(nxt >= _NBUF)
                def _():
                    wait_store(bn, nxt - _NBUF)

                issue_gather(bn, nxt)

        return carry

    lax.fori_loop(0, spw // _NBUF, outer, 0)
    for b in range(_NBUF):
        wait_store(b, spw - _NBUF + b)


def kernel(x, embedding_table):
    B, S = x.shape
    xf = x.reshape(B * S).astype(jnp.int32)
    pos = _positional_encoding()

    mesh = plsc.VectorSubcoreMesh(core_axis_name="c", subcore_axis_name="s")
    out = pl.kernel(
        _sc_body,
        out_type=jax.ShapeDtypeStruct((B, S, _D), jnp.float32),
        mesh=mesh,
        scratch_types=[
            pltpu.VMEM((_NBUF, _S), jnp.int32),
            pltpu.VMEM((_NBUF, _S, _D), jnp.float32),
            pltpu.VMEM((_S, _D), jnp.float32),
            pltpu.SemaphoreType.DMA((_NBUF,)),
            pltpu.SemaphoreType.DMA((_NBUF,)),
        ],
        compiler_params=pltpu.CompilerParams(use_tc_tiling_on_sc=False),
    )(xf, embedding_table, pos)
    return out
